# baseline (speedup 1.0000x reference)
import functools

import jax
import jax.numpy as jnp
from jax import lax
from jax.experimental import pallas as pl
from jax.experimental.pallas import tpu as pltpu
from jax.experimental.pallas import tpu_sc as plsc

_LOG1P_COEF = (
    3.910905549409094e-08, 0.9999936302585134, -0.4998254986434647,
    0.33144665224336606, -0.2394333707458602, 0.16499812983396112,
    -0.09229041738050231, 0.03426459995555095, -0.006006605050865348,
)

_CHUNK = 16000


def _sc_partials(x, t, m):
    info = plsc.get_sparse_core_info()
    nc, ns, L = info.num_cores, info.num_subcores, info.num_lanes
    nw = nc * ns
    per_w = x.size // nw
    nblk = per_w // _CHUNK
    nvec = _CHUNK // L
    mesh = plsc.VectorSubcoreMesh(core_axis_name="c", subcore_axis_name="s")

    @functools.partial(
        pl.kernel, mesh=mesh,
        compiler_params=pltpu.CompilerParams(needs_layout_passes=False),
        out_type=jax.ShapeDtypeStruct((nw, 20 * L), jnp.float32),
        scratch_types=[
            pltpu.VMEM((_CHUNK,), jnp.float32),
            pltpu.VMEM((_CHUNK,), jnp.float32),
            pltpu.VMEM((_CHUNK,), jnp.float32),
            pltpu.VMEM((20 * L,), jnp.float32),
        ],
    )
    def k(x_hbm, t_hbm, m_hbm, out_hbm, xv, tv, mv, acc):
        wid = lax.axis_index("s") * nc + lax.axis_index("c")
        base = wid * per_w
        zero = jnp.zeros((L,), jnp.float32)
        for r in range(20):
            acc[pl.ds(r * L, L)] = zero
        lanes = lax.iota(jnp.int32, L)

        def blk(j, carry):
            off = base + j * _CHUNK
            pltpu.sync_copy(x_hbm.at[pl.ds(off, _CHUNK)], xv)
            pltpu.sync_copy(t_hbm.at[pl.ds(off, _CHUNK)], tv)
            pltpu.sync_copy(m_hbm.at[pl.ds(off, _CHUNK)], mv)

            def inner(i, c):
                s = i * L
                xx = xv[pl.ds(s, L)]
                tt = tv[pl.ds(s, L)]
                mm = mv[pl.ds(s, L)]
                y = xx * (1.0 - 2.0 * tt)
                E = jnp.exp(-jnp.abs(y))
                p = jnp.full((L,), _LOG1P_COEF[-1], jnp.float32)
                for cf in _LOG1P_COEF[-2::-1]:
                    p = p * E + cf
                per = jnp.maximum(y, 0.0) + p
                w = E / (1.0 + E)
                g = jnp.where(y >= 0.0, 1.0 - w, w)
                b = jnp.minimum(g * 10.0, 9.5).astype(jnp.int32)
                v01 = jnp.where(mm > 0.0, 1.0, 0.0)
                idx = b * L + lanes
                plsc.addupdate_scatter(acc, [idx], v01)
                plsc.addupdate_scatter(acc, [idx + 10 * L], per * v01)
                return c

            lax.fori_loop(0, nvec, inner, 0)
            return carry

        lax.fori_loop(0, nblk, blk, 0)
        pltpu.sync_copy(acc, out_hbm.at[wid])

    return k(x, t, m)


def _combine(parts):
    rows, L = parts.shape

    def ck(p_ref, o_ref):
        arr = p_ref[...]
        rb = lax.broadcasted_iota(jnp.int32, (rows, L), 0) % 20
        tsum = jnp.float32(0.0)
        nn = jnp.float32(0.0)
        for b in range(10):
            cb = jnp.sum(jnp.where(rb == b, arr, 0.0))
            sb = jnp.sum(jnp.where(rb == b + 10, arr, 0.0))
            tsum += sb / jnp.maximum(cb, 1.0)
            nn += jnp.where(cb > 0.0, 1.0, 0.0)
        o_ref[...] = jnp.full((8, 128), tsum / jnp.maximum(nn, 1.0), jnp.float32)

    return pl.pallas_call(
        ck, out_shape=jax.ShapeDtypeStruct((8, 128), jnp.float32),
    )(parts)


def kernel(input, target, mask):
    x = input.reshape(-1)
    t = target.reshape(-1)
    m = mask.reshape(-1)
    parts = _sc_partials(x, t, m)
    out = _combine(parts.reshape(-1, 16))
    return out[0, 0]


# speedup vs baseline: 1.0611x; 1.0611x over previous
import functools

import jax
import jax.numpy as jnp
from jax import lax
from jax.experimental import pallas as pl
from jax.experimental.pallas import tpu as pltpu
from jax.experimental.pallas import tpu_sc as plsc

_LOG1P_COEF = (
    3.910905549409094e-08, 0.9999936302585134, -0.4998254986434647,
    0.33144665224336606, -0.2394333707458602, 0.16499812983396112,
    -0.09229041738050231, 0.03426459995555095, -0.006006605050865348,
)

_CHUNK = 16000


def _sc_partials(x, t, m):
    info = plsc.get_sparse_core_info()
    nc, ns, L = info.num_cores, info.num_subcores, info.num_lanes
    nw = nc * ns
    per_w = x.size // nw
    nblk = per_w // _CHUNK
    nvec = _CHUNK // L
    mesh = plsc.VectorSubcoreMesh(core_axis_name="c", subcore_axis_name="s")

    @functools.partial(
        pl.kernel, mesh=mesh,
        compiler_params=pltpu.CompilerParams(needs_layout_passes=False),
        out_type=jax.ShapeDtypeStruct((nw, 20 * L), jnp.float32),
        scratch_types=[
            pltpu.VMEM((2, _CHUNK), jnp.float32),
            pltpu.VMEM((2, _CHUNK), jnp.float32),
            pltpu.VMEM((2, _CHUNK), jnp.float32),
            pltpu.VMEM((20 * L,), jnp.float32),
            pltpu.SemaphoreType.DMA,
            pltpu.SemaphoreType.DMA,
        ],
    )
    def k(x_hbm, t_hbm, m_hbm, out_hbm, xv, tv, mv, acc, sem0, sem1):
        wid = lax.axis_index("s") * nc + lax.axis_index("c")
        base = wid * per_w
        sems = (sem0, sem1)
        zero = jnp.zeros((L,), jnp.float32)
        for r in range(20):
            acc[pl.ds(r * L, L)] = zero
        lanes = lax.iota(jnp.int32, L)

        def dma(j, slot):
            off = base + j * _CHUNK
            return (
                pltpu.make_async_copy(x_hbm.at[pl.ds(off, _CHUNK)], xv.at[slot], sems[slot]),
                pltpu.make_async_copy(t_hbm.at[pl.ds(off, _CHUNK)], tv.at[slot], sems[slot]),
                pltpu.make_async_copy(m_hbm.at[pl.ds(off, _CHUNK)], mv.at[slot], sems[slot]),
            )

        def start(j, slot):
            for c in dma(j, slot):
                c.start()

        def wait(j, slot):
            for c in dma(j, slot):
                c.wait()

        def compute(slot):
            def inner(i, c):
                s = i * L
                xx = xv[slot, pl.ds(s, L)]
                tt = tv[slot, pl.ds(s, L)]
                mm = mv[slot, pl.ds(s, L)]
                y = xx * (1.0 - 2.0 * tt)
                E = jnp.exp(-jnp.abs(y))
                p = jnp.full((L,), _LOG1P_COEF[-1], jnp.float32)
                for cf in _LOG1P_COEF[-2::-1]:
                    p = p * E + cf
                per = jnp.maximum(y, 0.0) + p
                w = E / (1.0 + E)
                g = jnp.where(y >= 0.0, 1.0 - w, w)
                b = jnp.minimum(g * 10.0, 9.5).astype(jnp.int32)
                idx = b * L + lanes
                plsc.addupdate_scatter(acc, [idx], mm)
                plsc.addupdate_scatter(acc, [idx + 10 * L], per * mm)
                return c

            lax.fori_loop(0, nvec, inner, 0, unroll=8)

        start(0, 0)

        def outer(jj, carry):
            for b in range(2):
                j = jj * 2 + b

                @pl.when(j + 1 < nblk)
                def _():
                    start(j + 1, 1 - b)

                wait(j, b)
                compute(b)
            return carry

        lax.fori_loop(0, nblk // 2, outer, 0)
        pltpu.sync_copy(acc, out_hbm.at[wid])

    return k(x, t, m)


def _combine(parts):
    rows, L = parts.shape

    def ck(p_ref, o_ref):
        arr = p_ref[...]
        rb = lax.broadcasted_iota(jnp.int32, (rows, L), 0) % 20
        tsum = jnp.float32(0.0)
        nn = jnp.float32(0.0)
        for b in range(10):
            cb = jnp.sum(jnp.where(rb == b, arr, 0.0))
            sb = jnp.sum(jnp.where(rb == b + 10, arr, 0.0))
            tsum += sb / jnp.maximum(cb, 1.0)
            nn += jnp.where(cb > 0.0, 1.0, 0.0)
        o_ref[...] = jnp.full((8, 128), tsum / jnp.maximum(nn, 1.0), jnp.float32)

    return pl.pallas_call(
        ck, out_shape=jax.ShapeDtypeStruct((8, 128), jnp.float32),
    )(parts)


def kernel(input, target, mask):
    x = input.reshape(-1)
    t = target.reshape(-1)
    m = mask.reshape(-1)
    parts = _sc_partials(x, t, m)
    out = _combine(parts.reshape(-1, 16))
    return out[0, 0]


# speedup vs baseline: 2.4830x; 2.3399x over previous
import functools

import jax
import jax.numpy as jnp
from jax import lax
from jax.experimental import pallas as pl
from jax.experimental.pallas import tpu as pltpu
from jax.experimental.pallas import tpu_sc as plsc

_LOG1P_COEF = (
    1.1447097560735031e-05, 0.9991664010110692, -0.48969909032083947,
    0.28382318306531834, -0.1299571976582333, 0.029808765243435193,
)
_CHUNK = 16000
_K = 8


def _sc_partials(x, t, m):
    info = plsc.get_sparse_core_info()
    nc, ns, L = info.num_cores, info.num_subcores, info.num_lanes
    nw = nc * ns
    per_w = x.size // nw
    nblk = per_w // _CHUNK
    nvec = _CHUNK // L
    mesh = plsc.VectorSubcoreMesh(core_axis_name="c", subcore_axis_name="s")

    @functools.partial(
        pl.kernel, mesh=mesh,
        compiler_params=pltpu.CompilerParams(needs_layout_passes=False),
        out_type=jax.ShapeDtypeStruct((nw, 20 * L), jnp.float32),
        scratch_types=[
            pltpu.VMEM((2, _CHUNK), jnp.float32),
            pltpu.VMEM((2, _CHUNK), jnp.float32),
            pltpu.VMEM((2, _CHUNK), jnp.float32),
            pltpu.VMEM((20 * L,), jnp.float32),
            pltpu.SemaphoreType.DMA,
            pltpu.SemaphoreType.DMA,
        ],
    )
    def k(x_hbm, t_hbm, m_hbm, out_hbm, xv, tv, mv, acc, sem0, sem1):
        wid = lax.axis_index("s") * nc + lax.axis_index("c")
        base = wid * per_w
        sems = (sem0, sem1)
        zero = jnp.zeros((L,), jnp.float32)
        for r in range(20):
            acc[pl.ds(r * L, L)] = zero
        lanes = lax.iota(jnp.int32, L)

        def dma(j, slot):
            off = base + j * _CHUNK
            return (
                pltpu.make_async_copy(x_hbm.at[pl.ds(off, _CHUNK)], xv.at[slot], sems[slot]),
                pltpu.make_async_copy(t_hbm.at[pl.ds(off, _CHUNK)], tv.at[slot], sems[slot]),
                pltpu.make_async_copy(m_hbm.at[pl.ds(off, _CHUNK)], mv.at[slot], sems[slot]),
            )

        def start(j, slot):
            for c in dma(j, slot):
                c.start()

        def wait(j, slot):
            for c in dma(j, slot):
                c.wait()

        def compute(slot):
            def inner(i, c):
                s = i * (L * _K)
                xs = [xv[slot, pl.ds(s + kk * L, L)] for kk in range(_K)]
                ts = [tv[slot, pl.ds(s + kk * L, L)] for kk in range(_K)]
                ms = [mv[slot, pl.ds(s + kk * L, L)] for kk in range(_K)]
                res = []
                for kk in range(_K):
                    xx, tt, mm = xs[kk], ts[kk], ms[kk]
                    ay = jnp.abs(xx)
                    E = jnp.exp(-ay)
                    p = jnp.full((L,), _LOG1P_COEF[-1], jnp.float32)
                    for cf in _LOG1P_COEF[-2::-1]:
                        p = p * E + cf
                    ypos = (xx >= 0.0) ^ (tt > 0.0)
                    per = jnp.where(ypos, ay, 0.0) + p
                    w = E / (1.0 + E)
                    g = jnp.where(ypos, 1.0 - w, w)
                    b = jnp.minimum(g * 10.0, 9.5).astype(jnp.int32)
                    idx = b * L + lanes
                    res.append((idx, mm, per * mm))
                for idx, mm, sper in res:
                    plsc.addupdate_scatter(acc, [idx], mm)
                    plsc.addupdate_scatter(acc, [idx + 10 * L], sper)
                return c

            lax.fori_loop(0, nvec // _K, inner, 0)

        start(0, 0)

        def outer(jj, carry):
            for b in range(2):
                j = jj * 2 + b

                @pl.when(j + 1 < nblk)
                def _():
                    start(j + 1, 1 - b)

                wait(j, b)
                compute(b)
            return carry

        lax.fori_loop(0, nblk // 2, outer, 0)
        pltpu.sync_copy(acc, out_hbm.at[wid])

    return k(x, t, m)


def _combine(parts):
    rows, L = parts.shape

    def ck(p_ref, o_ref):
        arr = p_ref[...]
        rb = lax.broadcasted_iota(jnp.int32, (rows, L), 0) % 20
        tsum = jnp.float32(0.0)
        nn = jnp.float32(0.0)
        for b in range(10):
            cb = jnp.sum(jnp.where(rb == b, arr, 0.0))
            sb = jnp.sum(jnp.where(rb == b + 10, arr, 0.0))
            tsum += sb / jnp.maximum(cb, 1.0)
            nn += jnp.where(cb > 0.0, 1.0, 0.0)
        o_ref[...] = jnp.full((8, 128), tsum / jnp.maximum(nn, 1.0), jnp.float32)

    return pl.pallas_call(
        ck, out_shape=jax.ShapeDtypeStruct((8, 128), jnp.float32),
    )(parts)


def kernel(input, target, mask):
    x = input.reshape(-1)
    t = target.reshape(-1)
    m = mask.reshape(-1)
    parts = _sc_partials(x, t, m)
    out = _combine(parts.reshape(-1, 16))
    return out[0, 0]


# speedup vs baseline: 2.6245x; 1.0570x over previous
import functools

import jax
import jax.numpy as jnp
from jax import lax
from jax.experimental import pallas as pl
from jax.experimental.pallas import tpu as pltpu
from jax.experimental.pallas import tpu_sc as plsc

_LOG1P_COEF = (
    7.942077648770418e-05, 0.9959657831345109, -0.4650204374456057,
    0.2164487077843725, -0.054370933555584255,
)
_CHUNK = 16000
_K = 8


def _sc_partials(x, t, m):
    info = plsc.get_sparse_core_info()
    nc, ns, L = info.num_cores, info.num_subcores, info.num_lanes
    nw = nc * ns
    per_w = x.size // nw
    nblk = per_w // _CHUNK
    nvec = _CHUNK // L
    mesh = plsc.VectorSubcoreMesh(core_axis_name="c", subcore_axis_name="s")

    @functools.partial(
        pl.kernel, mesh=mesh,
        compiler_params=pltpu.CompilerParams(needs_layout_passes=False),
        out_type=jax.ShapeDtypeStruct((nw, 20 * L), jnp.float32),
        scratch_types=[
            pltpu.VMEM((2, _CHUNK), jnp.float32),
            pltpu.VMEM((2, _CHUNK), jnp.float32),
            pltpu.VMEM((2, _CHUNK), jnp.float32),
            pltpu.VMEM((20 * L,), jnp.float32),
            pltpu.SemaphoreType.DMA,
            pltpu.SemaphoreType.DMA,
        ],
    )
    def k(x_hbm, t_hbm, m_hbm, out_hbm, xv, tv, mv, acc, sem0, sem1):
        wid = lax.axis_index("s") * nc + lax.axis_index("c")
        base = wid * per_w
        sems = (sem0, sem1)
        zero = jnp.zeros((L,), jnp.float32)
        for r in range(20):
            acc[pl.ds(r * L, L)] = zero
        lanes = lax.iota(jnp.int32, L)

        def dma(j, slot):
            off = base + j * _CHUNK
            return (
                pltpu.make_async_copy(x_hbm.at[pl.ds(off, _CHUNK)], xv.at[slot], sems[slot]),
                pltpu.make_async_copy(t_hbm.at[pl.ds(off, _CHUNK)], tv.at[slot], sems[slot]),
                pltpu.make_async_copy(m_hbm.at[pl.ds(off, _CHUNK)], mv.at[slot], sems[slot]),
            )

        def start(j, slot):
            for c in dma(j, slot):
                c.start()

        def wait(j, slot):
            for c in dma(j, slot):
                c.wait()

        def compute(slot):
            def inner(i, c):
                s = i * (L * _K)
                xs = [xv[slot, pl.ds(s + kk * L, L)] for kk in range(_K)]
                ts = [tv[slot, pl.ds(s + kk * L, L)] for kk in range(_K)]
                ms = [mv[slot, pl.ds(s + kk * L, L)] for kk in range(_K)]
                res = []
                for kk in range(_K):
                    xx, tt, mm = xs[kk], ts[kk], ms[kk]
                    ay = jnp.abs(xx)
                    E = jnp.exp(-ay)
                    p = jnp.full((L,), _LOG1P_COEF[-1], jnp.float32)
                    for cf in _LOG1P_COEF[-2::-1]:
                        p = p * E + cf
                    ypos = (xx >= 0.0) ^ (tt > 0.0)
                    per = jnp.where(ypos, ay, 0.0) + p
                    q = 1.0 / (1.0 + E)
                    g = jnp.where(ypos, q, 1.0 - q)
                    b = jnp.minimum(g * 10.0, 9.5).astype(jnp.int32)
                    idx = b * L + lanes
                    res.append((idx, mm, per * mm))
                for idx, mm, sper in res:
                    plsc.addupdate_scatter(acc, [idx], mm)
                    plsc.addupdate_scatter(acc, [idx + 10 * L], sper)
                return c

            lax.fori_loop(0, nvec // _K, inner, 0)

        start(0, 0)

        def outer(jj, carry):
            for b in range(2):
                j = jj * 2 + b

                @pl.when(j + 1 < nblk)
                def _():
                    start(j + 1, 1 - b)

                wait(j, b)
                compute(b)
            return carry

        lax.fori_loop(0, nblk // 2, outer, 0)
        pltpu.sync_copy(acc, out_hbm.at[wid])

    return k(x, t, m)


def _combine(parts):
    rows, L = parts.shape

    def ck(p_ref, o_ref):
        arr = p_ref[...]
        rb = lax.broadcasted_iota(jnp.int32, (rows, L), 0) % 20
        tsum = jnp.float32(0.0)
        nn = jnp.float32(0.0)
        for b in range(10):
            cb = jnp.sum(jnp.where(rb == b, arr, 0.0))
            sb = jnp.sum(jnp.where(rb == b + 10, arr, 0.0))
            tsum += sb / jnp.maximum(cb, 1.0)
            nn += jnp.where(cb > 0.0, 1.0, 0.0)
        o_ref[...] = jnp.full((8, 128), tsum / jnp.maximum(nn, 1.0), jnp.float32)

    return pl.pallas_call(
        ck, out_shape=jax.ShapeDtypeStruct((8, 128), jnp.float32),
    )(parts)


def kernel(input, target, mask):
    x = input.reshape(-1)
    t = target.reshape(-1)
    m = mask.reshape(-1)
    parts = _sc_partials(x, t, m)
    out = _combine(parts.reshape(-1, 16))
    return out[0, 0]


# speedup vs baseline: 2.8988x; 1.1045x over previous
import functools

import jax
import jax.numpy as jnp
from jax import lax
from jax.experimental import pallas as pl
from jax.experimental.pallas import tpu as pltpu
from jax.experimental.pallas import tpu_sc as plsc

_LOG1P_COEF = (
    7.942077648770418e-05, 0.9959657831345109, -0.4650204374456057,
    0.2164487077843725, -0.054370933555584255,
)
_CHUNK = 16000
_K = 8

_TC_ROWS = 8192
_TC_BLK = 512


def _sc_partials(x, t, m, start_elem):
    info = plsc.get_sparse_core_info()
    nc, ns, L = info.num_cores, info.num_subcores, info.num_lanes
    nw = nc * ns
    per_w = (x.size - start_elem) // nw
    nblk = per_w // _CHUNK
    nvec = _CHUNK // L
    mesh = plsc.VectorSubcoreMesh(core_axis_name="c", subcore_axis_name="s")

    @functools.partial(
        pl.kernel, mesh=mesh,
        compiler_params=pltpu.CompilerParams(needs_layout_passes=False),
        out_type=jax.ShapeDtypeStruct((nw, 20 * L), jnp.float32),
        scratch_types=[
            pltpu.VMEM((2, _CHUNK), jnp.float32),
            pltpu.VMEM((2, _CHUNK), jnp.float32),
            pltpu.VMEM((2, _CHUNK), jnp.float32),
            pltpu.VMEM((20 * L,), jnp.float32),
            pltpu.SemaphoreType.DMA,
            pltpu.SemaphoreType.DMA,
        ],
    )
    def k(x_hbm, t_hbm, m_hbm, out_hbm, xv, tv, mv, acc, sem0, sem1):
        wid = lax.axis_index("s") * nc + lax.axis_index("c")
        base = start_elem + wid * per_w
        sems = (sem0, sem1)
        zero = jnp.zeros((L,), jnp.float32)
        for r in range(20):
            acc[pl.ds(r * L, L)] = zero
        lanes = lax.iota(jnp.int32, L)

        def dma(j, slot):
            off = base + j * _CHUNK
            return (
                pltpu.make_async_copy(x_hbm.at[pl.ds(off, _CHUNK)], xv.at[slot], sems[slot]),
                pltpu.make_async_copy(t_hbm.at[pl.ds(off, _CHUNK)], tv.at[slot], sems[slot]),
                pltpu.make_async_copy(m_hbm.at[pl.ds(off, _CHUNK)], mv.at[slot], sems[slot]),
            )

        def start(j, slot):
            for c in dma(j, slot):
                c.start()

        def wait(j, slot):
            for c in dma(j, slot):
                c.wait()

        def compute(slot):
            def inner(i, c):
                s = i * (L * _K)
                xs = [xv[slot, pl.ds(s + kk * L, L)] for kk in range(_K)]
                ts = [tv[slot, pl.ds(s + kk * L, L)] for kk in range(_K)]
                ms = [mv[slot, pl.ds(s + kk * L, L)] for kk in range(_K)]
                res = []
                for kk in range(_K):
                    xx, tt, mm = xs[kk], ts[kk], ms[kk]
                    ay = jnp.abs(xx)
                    E = jnp.exp(-ay)
                    p = jnp.full((L,), _LOG1P_COEF[-1], jnp.float32)
                    for cf in _LOG1P_COEF[-2::-1]:
                        p = p * E + cf
                    ypos = (xx >= 0.0) ^ (tt > 0.0)
                    per = jnp.where(ypos, ay, 0.0) + p
                    q = 1.0 / (1.0 + E)
                    g = jnp.where(ypos, q, 1.0 - q)
                    b = jnp.minimum(g * 10.0, 9.5).astype(jnp.int32)
                    idx = b * L + lanes
                    res.append((idx, mm, per * mm))
                for idx, mm, sper in res:
                    plsc.addupdate_scatter(acc, [idx], mm)
                    plsc.addupdate_scatter(acc, [idx + 10 * L], sper)
                return c

            lax.fori_loop(0, nvec // _K, inner, 0)

        start(0, 0)

        def outer(jj, carry):
            for b in range(2):
                j = jj * 2 + b

                @pl.when(j + 1 < nblk)
                def _():
                    start(j + 1, 1 - b)

                wait(j, b)
                compute(b)
            return carry

        lax.fori_loop(0, nblk // 2, outer, 0)
        pltpu.sync_copy(acc, out_hbm.at[wid])

    return k(x, t, m)


def _tc_partials(x, t, m, tc_rows):
    cols = x.shape[1]
    grid = tc_rows // _TC_BLK

    def body(x_ref, t_ref, m_ref, o_ref):
        xx = x_ref[...]
        tt = t_ref[...]
        mm = m_ref[...]
        g = jnp.abs(jax.nn.sigmoid(xx) - tt)
        per = jnp.maximum(xx, 0.0) - xx * tt + jnp.log1p(jnp.exp(-jnp.abs(xx)))
        bi = jnp.minimum(g * 10.0, 9.5).astype(jnp.int32)
        sper = per * mm
        vals = []
        for b in range(10):
            mb = bi == b
            vals.append(jnp.sum(jnp.where(mb, mm, 0.0)))
        for b in range(10):
            mb = bi == b
            vals.append(jnp.sum(jnp.where(mb, sper, 0.0)))
        vals += [jnp.float32(0.0)] * 4
        v24 = jnp.stack(vals).reshape(24, 1)
        lane = lax.broadcasted_iota(jnp.int32, (24, 128), 1)
        o_ref[...] = jnp.where(lane == 0, v24, 0.0).reshape(1, 24, 128)

    return pl.pallas_call(
        body,
        grid=(grid,),
        in_specs=[
            pl.BlockSpec((_TC_BLK, cols), lambda i: (i, 0)),
            pl.BlockSpec((_TC_BLK, cols), lambda i: (i, 0)),
            pl.BlockSpec((_TC_BLK, cols), lambda i: (i, 0)),
        ],
        out_specs=pl.BlockSpec((1, 24, 128), lambda i: (i, 0, 0)),
        out_shape=jax.ShapeDtypeStruct((grid, 24, 128), jnp.float32),
    )(x, t, m)


def _combine(sc_parts, tc_parts):
    sc_rows, scl = sc_parts.shape
    tc_rows_, tcl = tc_parts.shape

    def ck(sc_ref, tc_ref, o_ref):
        sc = sc_ref[...]
        tc = tc_ref[...]
        rb_sc = lax.broadcasted_iota(jnp.int32, (sc_rows, scl), 0) % 20
        rb_tc = lax.broadcasted_iota(jnp.int32, (tc_rows_, tcl), 0) % 24
        tsum = jnp.float32(0.0)
        nn = jnp.float32(0.0)
        for b in range(10):
            cb = (jnp.sum(jnp.where(rb_sc == b, sc, 0.0))
                  + jnp.sum(jnp.where(rb_tc == b, tc, 0.0)))
            sb = (jnp.sum(jnp.where(rb_sc == b + 10, sc, 0.0))
                  + jnp.sum(jnp.where(rb_tc == b + 10, tc, 0.0)))
            tsum += sb / jnp.maximum(cb, 1.0)
            nn += jnp.where(cb > 0.0, 1.0, 0.0)
        o_ref[...] = jnp.full((8, 128), tsum / jnp.maximum(nn, 1.0), jnp.float32)

    return pl.pallas_call(
        ck, out_shape=jax.ShapeDtypeStruct((8, 128), jnp.float32),
    )(sc_parts, tc_parts)


def kernel(input, target, mask):
    cols = input.shape[1]
    tc_parts = _tc_partials(input, target, mask, _TC_ROWS)
    x = input.reshape(-1)
    t = target.reshape(-1)
    m = mask.reshape(-1)
    sc_parts = _sc_partials(x, t, m, _TC_ROWS * cols)
    out = _combine(sc_parts.reshape(-1, 16), tc_parts.reshape(-1, 128))
    return out[0, 0]


# speedup vs baseline: 5.1139x; 1.7641x over previous
import functools

import jax
import jax.numpy as jnp
from jax import lax
from jax.experimental import pallas as pl
from jax.experimental.pallas import tpu as pltpu
from jax.experimental.pallas import tpu_sc as plsc

_LOG1P_COEF = (
    7.942077648770418e-05, 0.9959657831345109, -0.4650204374456057,
    0.2164487077843725, -0.054370933555584255,
)
_CHUNK = 16000
_K = 8

_TC_ROWS = 8192
_TC_BLK = 512


_ROWS_BLK = 8


def _sc_partials(x, t, m, start_row):
    info = plsc.get_sparse_core_info()
    nc, ns, L = info.num_cores, info.num_subcores, info.num_lanes
    nw = nc * ns
    nrows, cols = x.shape
    rows_w = (nrows - start_row) // nw
    nblk = rows_w // _ROWS_BLK
    ntile = cols // 128
    rem = cols - ntile * 128
    nfull = rem // L
    mesh = plsc.VectorSubcoreMesh(core_axis_name="c", subcore_axis_name="s")

    @functools.partial(
        pl.kernel, mesh=mesh,
        compiler_params=pltpu.CompilerParams(
            needs_layout_passes=False, use_tc_tiling_on_sc=True),
        out_type=jax.ShapeDtypeStruct((nw, 20 * L), jnp.float32),
        scratch_types=[
            pltpu.VMEM((2, _ROWS_BLK, cols), jnp.float32),
            pltpu.VMEM((2, _ROWS_BLK, cols), jnp.float32),
            pltpu.VMEM((2, _ROWS_BLK, cols), jnp.float32),
            pltpu.VMEM((20 * L,), jnp.float32),
            pltpu.SemaphoreType.DMA,
            pltpu.SemaphoreType.DMA,
        ],
    )
    def k(x_hbm, t_hbm, m_hbm, out_hbm, xv, tv, mv, acc, sem0, sem1):
        wid = lax.axis_index("s") * nc + lax.axis_index("c")
        base = start_row + wid * rows_w
        sems = (sem0, sem1)
        zero = jnp.zeros((L,), jnp.float32)
        for r in range(20):
            acc[pl.ds(r * L, L)] = zero
        lanes = lax.iota(jnp.int32, L)
        ovmask = jnp.where(lanes >= (L - rem % L), 1.0, 0.0)

        def dma(j, slot):
            r0 = base + j * _ROWS_BLK
            return (
                pltpu.make_async_copy(x_hbm.at[pl.ds(r0, _ROWS_BLK), :], xv.at[slot], sems[slot]),
                pltpu.make_async_copy(t_hbm.at[pl.ds(r0, _ROWS_BLK), :], tv.at[slot], sems[slot]),
                pltpu.make_async_copy(m_hbm.at[pl.ds(r0, _ROWS_BLK), :], mv.at[slot], sems[slot]),
            )

        def start(j, slot):
            for c in dma(j, slot):
                c.start()

        def wait(j, slot):
            for c in dma(j, slot):
                c.wait()

        def batch(slot, r, offs, masks):
            xs = [xv[slot, r, pl.ds(c, L)] for c in offs]
            ts = [tv[slot, r, pl.ds(c, L)] for c in offs]
            ms = [mv[slot, r, pl.ds(c, L)] for c in offs]
            res = []
            for kk in range(len(offs)):
                xx, tt, mm = xs[kk], ts[kk], ms[kk]
                if masks[kk] is not None:
                    mm = mm * masks[kk]
                ay = jnp.abs(xx)
                E = jnp.exp(-ay)
                p = jnp.full((L,), _LOG1P_COEF[-1], jnp.float32)
                for cf in _LOG1P_COEF[-2::-1]:
                    p = p * E + cf
                ypos = (xx >= 0.0) ^ (tt > 0.0)
                per = jnp.where(ypos, ay, 0.0) + p
                q = 1.0 / (1.0 + E)
                g = jnp.where(ypos, q, 1.0 - q)
                b = jnp.minimum(g * 10.0, 9.5).astype(jnp.int32)
                idx = b * L + lanes
                res.append((idx, mm, per * mm))
            for idx, mm, sper in res:
                plsc.addupdate_scatter(acc, [idx], mm)
                plsc.addupdate_scatter(acc, [idx + 10 * L], sper)

        def compute(slot):
            def row_body(r, carry):
                def grp(cg, c2):
                    s = cg * 128
                    batch(slot, r, [s + kk * L for kk in range(_K)], [None] * _K)
                    return c2

                lax.fori_loop(0, ntile, grp, 0)
                offs = [ntile * 128 + kk * L for kk in range(nfull)]
                masks = [None] * nfull
                if rem % L:
                    offs.append(cols - L)
                    masks.append(ovmask)
                batch(slot, r, offs, masks)
                return carry

            lax.fori_loop(0, _ROWS_BLK, row_body, 0)

        start(0, 0)

        def outer(jj, carry):
            for b in range(2):
                j = jj * 2 + b

                @pl.when(j + 1 < nblk)
                def _():
                    start(j + 1, 1 - b)

                wait(j, b)
                compute(b)
            return carry

        lax.fori_loop(0, nblk // 2, outer, 0)
        pltpu.sync_copy(acc, out_hbm.at[wid])

    return k(x, t, m)


def _tc_partials(x, t, m, tc_rows):
    cols = x.shape[1]
    grid = tc_rows // _TC_BLK

    def body(x_ref, t_ref, m_ref, o_ref):
        xx = x_ref[...]
        tt = t_ref[...]
        mm = m_ref[...]
        g = jnp.abs(jax.nn.sigmoid(xx) - tt)
        per = jnp.maximum(xx, 0.0) - xx * tt + jnp.log1p(jnp.exp(-jnp.abs(xx)))
        bi = jnp.minimum(g * 10.0, 9.5).astype(jnp.int32)
        sper = per * mm
        vals = []
        for b in range(10):
            mb = bi == b
            vals.append(jnp.sum(jnp.where(mb, mm, 0.0)))
        for b in range(10):
            mb = bi == b
            vals.append(jnp.sum(jnp.where(mb, sper, 0.0)))
        vals += [jnp.float32(0.0)] * 4
        v24 = jnp.stack(vals).reshape(24, 1)
        lane = lax.broadcasted_iota(jnp.int32, (24, 128), 1)
        o_ref[...] = jnp.where(lane == 0, v24, 0.0).reshape(1, 24, 128)

    return pl.pallas_call(
        body,
        grid=(grid,),
        in_specs=[
            pl.BlockSpec((_TC_BLK, cols), lambda i: (i, 0)),
            pl.BlockSpec((_TC_BLK, cols), lambda i: (i, 0)),
            pl.BlockSpec((_TC_BLK, cols), lambda i: (i, 0)),
        ],
        out_specs=pl.BlockSpec((1, 24, 128), lambda i: (i, 0, 0)),
        out_shape=jax.ShapeDtypeStruct((grid, 24, 128), jnp.float32),
    )(x, t, m)


def _combine(sc_parts, tc_parts):
    sc_rows, scl = sc_parts.shape
    tc_rows_, tcl = tc_parts.shape

    def ck(sc_ref, tc_ref, o_ref):
        sc = sc_ref[...]
        tc = tc_ref[...]
        rb_sc = lax.broadcasted_iota(jnp.int32, (sc_rows, scl), 0) % 20
        rb_tc = lax.broadcasted_iota(jnp.int32, (tc_rows_, tcl), 0) % 24
        tsum = jnp.float32(0.0)
        nn = jnp.float32(0.0)
        for b in range(10):
            cb = (jnp.sum(jnp.where(rb_sc == b, sc, 0.0))
                  + jnp.sum(jnp.where(rb_tc == b, tc, 0.0)))
            sb = (jnp.sum(jnp.where(rb_sc == b + 10, sc, 0.0))
                  + jnp.sum(jnp.where(rb_tc == b + 10, tc, 0.0)))
            tsum += sb / jnp.maximum(cb, 1.0)
            nn += jnp.where(cb > 0.0, 1.0, 0.0)
        o_ref[...] = jnp.full((8, 128), tsum / jnp.maximum(nn, 1.0), jnp.float32)

    return pl.pallas_call(
        ck, out_shape=jax.ShapeDtypeStruct((8, 128), jnp.float32),
    )(sc_parts, tc_parts)


def kernel(input, target, mask):
    tc_parts = _tc_partials(input, target, mask, _TC_ROWS)
    sc_parts = _sc_partials(input, target, mask, _TC_ROWS)
    out = _combine(sc_parts.reshape(-1, 16), tc_parts.reshape(-1, 128))
    return out[0, 0]
